# baseline (speedup 1.0000x reference)
import jax
import jax.numpy as jnp
from jax.experimental import pallas as pl
from jax.experimental.pallas import tpu as pltpu

_CHUNKS = 4


def _dma_copy_kernel(x_ref, o_ref, sems):
    B = x_ref.shape[0]
    rows = B // _CHUNKS
    copies = [
        pltpu.make_async_copy(
            x_ref.at[pl.ds(i * rows, rows)],
            o_ref.at[pl.ds(i * rows, rows)],
            sems.at[i],
        )
        for i in range(_CHUNKS)
    ]
    for c in copies:
        c.start()
    for c in copies:
        c.wait()


def kernel(batch):
    return pl.pallas_call(
        _dma_copy_kernel,
        in_specs=[pl.BlockSpec(memory_space=pl.ANY)],
        out_specs=pl.BlockSpec(memory_space=pl.ANY),
        out_shape=jax.ShapeDtypeStruct(batch.shape, batch.dtype),
        scratch_shapes=[pltpu.SemaphoreType.DMA((_CHUNKS,))],
    )(batch)


# speedup vs baseline: 41.5340x; 41.5340x over previous
import jax
import jax.numpy as jnp
from jax.experimental import pallas as pl
from jax.experimental.pallas import tpu as pltpu


def _copy_kernel(x_ref, o_ref):
    o_ref[...] = x_ref[...]


def kernel(batch):
    B, F = batch.shape
    blk = 1024
    return pl.pallas_call(
        _copy_kernel,
        grid=(B // blk,),
        in_specs=[pl.BlockSpec((blk, F), lambda i: (i, 0))],
        out_specs=pl.BlockSpec((blk, F), lambda i: (i, 0)),
        out_shape=jax.ShapeDtypeStruct((B, F), batch.dtype),
        compiler_params=pltpu.CompilerParams(
            dimension_semantics=("arbitrary",),
        ),
    )(batch)


# speedup vs baseline: 47.3322x; 1.1396x over previous
import jax
import jax.numpy as jnp
from jax.experimental import pallas as pl
from jax.experimental.pallas import tpu as pltpu


def _copy_kernel(x_ref, o_ref):
    o_ref[...] = x_ref[...]


def kernel(batch):
    B, F = batch.shape
    blk = 2048
    return pl.pallas_call(
        _copy_kernel,
        grid=(B // blk,),
        in_specs=[pl.BlockSpec((blk, F), lambda i: (i, 0))],
        out_specs=pl.BlockSpec((blk, F), lambda i: (i, 0)),
        out_shape=jax.ShapeDtypeStruct((B, F), batch.dtype),
        compiler_params=pltpu.CompilerParams(
            dimension_semantics=("arbitrary",),
        ),
    )(batch)
